# baseline (speedup 1.0000x reference)
import functools

import jax
import jax.numpy as jnp
from jax import lax
from jax.experimental import pallas as pl
from jax.experimental.pallas import tpu as pltpu
from jax.experimental.pallas import tpu_sc as plsc

_L = 16
_NC = 2
_NS = 16
_NW = _NC * _NS
_CH = 16384
_SBW = 128


@functools.lru_cache(maxsize=None)
def _build(nnz: int, n: int):
    assert nnz % (_NW * _CH) == 0
    assert n & (n - 1) == 0
    shift = n.bit_length() - 1
    per_w = nnz // _NW
    nch = per_w // _CH
    cap = _CH + _SBW + _L

    mesh = plsc.VectorSubcoreMesh(
        core_axis_name="c", subcore_axis_name="s",
        num_cores=_NC, num_subcores=_NS)

    @functools.partial(
        pl.kernel,
        out_type=(),
        mesh=mesh,
        compiler_params=pltpu.CompilerParams(needs_layout_passes=False),
        scratch_types=[
            pltpu.VMEM((_CH,), jnp.int32),
            pltpu.VMEM((_CH,), jnp.int32),
            pltpu.VMEM((_CH,), jnp.int32),
            pltpu.VMEM((_CH,), jnp.int32),
            pltpu.VMEM((_CH,), jnp.float32),
            pltpu.VMEM((_CH,), jnp.float32),
            pltpu.VMEM((cap,), jnp.int32),
            pltpu.VMEM((_SBW,), jnp.float32),
            pltpu.SemaphoreType.DMA,
            pltpu.SemaphoreType.DMA,
            pltpu.SemaphoreType.DMA,
        ],
    )
    def scatter_zeros(out_hbm, rows_hbm, cols_hbm, samp_hbm,
                      rows_v0, rows_v1, cols_v0, cols_v1, samp_v0, samp_v1,
                      zidx_v, zeros_v, lsem, ssem0, ssem1):
        bufs = ((rows_v0, cols_v0, samp_v0), (rows_v1, cols_v1, samp_v1))
        wid = lax.axis_index("s") * _NC + lax.axis_index("c")

        for b in range(_SBW // _L):
            zeros_v[pl.ds(b * _L, _L)] = jnp.zeros((_L,), jnp.float32)

        def load_chunk(ch, buf):
            base = wid * per_w + ch * _CH
            r_v, c_v, s_v = bufs[buf]
            pltpu.async_copy(rows_hbm.at[pl.ds(base, _CH)], r_v, lsem)
            pltpu.async_copy(cols_hbm.at[pl.ds(base, _CH)], c_v, lsem)
            pltpu.async_copy(samp_hbm.at[pl.ds(base, _CH)], s_v, lsem)

        def wait_chunk(buf):
            r_v, c_v, s_v = bufs[buf]
            pltpu.make_async_copy(rows_hbm.at[pl.ds(0, _CH)], r_v, lsem).wait()
            pltpu.make_async_copy(cols_hbm.at[pl.ds(0, _CH)], c_v, lsem).wait()
            pltpu.make_async_copy(samp_hbm.at[pl.ds(0, _CH)], s_v, lsem).wait()

        def fire(j, sem):
            pltpu.async_copy(
                zeros_v, out_hbm.at[zidx_v.at[pl.ds(j * _SBW, _SBW)]], sem)

        def drain1(sem):
            pltpu.make_async_copy(
                zeros_v, out_hbm.at[zidx_v.at[pl.ds(0, _SBW)]], sem).wait()

        load_chunk(0, 0)

        for ch in range(nch):
            buf = ch & 1
            wait_chunk(buf)
            if ch + 1 < nch:
                load_chunk(ch + 1, 1 - buf)

            rows_b, cols_b, samp_b = bufs[buf]

            @plsc.parallel_loop(0, _CH, _L, unroll=8,
                                carry=jnp.zeros((_L,), jnp.int32))
            def cnt_vec(g, cv):
                r = rows_b[pl.ds(g, _L)]
                c = cols_b[pl.ds(g, _L)]
                s = samp_b[pl.ds(g, _L)]
                flat = (r << shift) | c
                m = s == 0.0
                pos = plsc.cumsum(m.astype(jnp.int32))
                plsc.store_scatter(zidx_v, [cv + pos - 1], flat, mask=m)
                return cv + plsc.all_reduce_population_count(m)

            cnt = cnt_vec[0]

            @pl.when(cnt > 0)
            def _():
                pad = jnp.full((_L,), zidx_v[pl.ds(0, _L)][0], jnp.int32)
                for k in range(_SBW // _L):
                    zidx_v[pl.ds(cnt + k * _L, _L)] = pad
                nb = (cnt + _SBW - 1) // _SBW
                nb2 = nb // 2

                def blk2(j, c2):
                    fire(2 * j, ssem0)
                    fire(2 * j + 1, ssem1)
                    return c2

                lax.fori_loop(0, nb2, blk2, 0)

                @pl.when(nb % 2 == 1)
                def _():
                    fire(nb - 1, ssem0)

                def drn2(j, c2):
                    drain1(ssem0)
                    drain1(ssem1)
                    return c2

                lax.fori_loop(0, nb2, drn2, 0)

                @pl.when(nb % 2 == 1)
                def _():
                    drain1(ssem0)

    return scatter_zeros


@functools.lru_cache(maxsize=None)
def _build_copy(n: int):
    blk = 256

    def body(x_ref, o_ref):
        o_ref[...] = x_ref[...]

    return pl.pallas_call(
        body,
        grid=(n // blk,),
        in_specs=[pl.BlockSpec((blk, n), lambda i: (i, 0))],
        out_specs=pl.BlockSpec((blk, n), lambda i: (i, 0)),
        out_shape=jax.ShapeDtypeStruct((n, n), jnp.float32),
    )


def kernel(input, new_sample, weight_rows, weight_cols, sample):
    n = input.shape[0]
    nnz = weight_rows.shape[0]
    out_ref = jax.new_ref(jnp.reshape(_build_copy(n)(input), (-1,)))
    _build(nnz, n)(out_ref, weight_rows, weight_cols, sample)
    return jnp.reshape(out_ref[...], (n, n))


# speedup vs baseline: 1.0710x; 1.0710x over previous
import functools

import jax
import jax.numpy as jnp
from jax import lax
from jax.experimental import pallas as pl
from jax.experimental.pallas import tpu as pltpu
from jax.experimental.pallas import tpu_sc as plsc

_L = 16
_NC = 2
_NS = 16
_NW = _NC * _NS
_CH = 16384
_SBW = 128


@functools.lru_cache(maxsize=None)
def _build(nnz: int, n: int):
    assert nnz % (_NW * _CH) == 0
    assert n & (n - 1) == 0
    shift = n.bit_length() - 1
    per_w = nnz // _NW
    nch = per_w // _CH
    cap = _CH + _SBW + _L

    mesh = plsc.VectorSubcoreMesh(
        core_axis_name="c", subcore_axis_name="s",
        num_cores=_NC, num_subcores=_NS)

    @functools.partial(
        pl.kernel,
        out_type=(),
        mesh=mesh,
        compiler_params=pltpu.CompilerParams(needs_layout_passes=False),
        scratch_types=[
            pltpu.VMEM((_CH,), jnp.int32),
            pltpu.VMEM((_CH,), jnp.int32),
            pltpu.VMEM((_CH,), jnp.int32),
            pltpu.VMEM((_CH,), jnp.int32),
            pltpu.VMEM((_CH,), jnp.float32),
            pltpu.VMEM((_CH,), jnp.float32),
            pltpu.VMEM((cap,), jnp.int32),
            pltpu.VMEM((_SBW,), jnp.float32),
            pltpu.SemaphoreType.DMA,
            pltpu.SemaphoreType.DMA,
            pltpu.SemaphoreType.DMA,
        ],
    )
    def scatter_zeros(out_hbm, rows_hbm, cols_hbm, samp_hbm,
                      rows_v0, rows_v1, cols_v0, cols_v1, samp_v0, samp_v1,
                      zidx_v, zeros_v, lsem, ssem0, ssem1):
        bufs = ((rows_v0, cols_v0, samp_v0), (rows_v1, cols_v1, samp_v1))
        wid = lax.axis_index("s") * _NC + lax.axis_index("c")

        for b in range(_SBW // _L):
            zeros_v[pl.ds(b * _L, _L)] = jnp.zeros((_L,), jnp.float32)

        def load_chunk(ch, buf):
            base = wid * per_w + ch * _CH
            r_v, c_v, s_v = bufs[buf]
            pltpu.async_copy(rows_hbm.at[pl.ds(base, _CH)], r_v, lsem)
            pltpu.async_copy(cols_hbm.at[pl.ds(base, _CH)], c_v, lsem)
            pltpu.async_copy(samp_hbm.at[pl.ds(base, _CH)], s_v, lsem)

        def wait_chunk(buf):
            r_v, c_v, s_v = bufs[buf]
            pltpu.make_async_copy(rows_hbm.at[pl.ds(0, _CH)], r_v, lsem).wait()
            pltpu.make_async_copy(cols_hbm.at[pl.ds(0, _CH)], c_v, lsem).wait()
            pltpu.make_async_copy(samp_hbm.at[pl.ds(0, _CH)], s_v, lsem).wait()

        def fire(j, sem):
            pltpu.async_copy(
                zeros_v, out_hbm.at[zidx_v.at[pl.ds(j * _SBW, _SBW)]], sem)

        def drain1(sem):
            pltpu.make_async_copy(
                zeros_v, out_hbm.at[zidx_v.at[pl.ds(0, _SBW)]], sem).wait()

        load_chunk(0, 0)

        for ch in range(nch):
            buf = ch & 1
            wait_chunk(buf)
            if ch + 1 < nch:
                load_chunk(ch + 1, 1 - buf)

            rows_b, cols_b, samp_b = bufs[buf]

            @plsc.parallel_loop(0, _CH, _L, unroll=8,
                                carry=jnp.zeros((_L,), jnp.int32))
            def cnt_vec(g, cv):
                r = rows_b[pl.ds(g, _L)]
                c = cols_b[pl.ds(g, _L)]
                s = samp_b[pl.ds(g, _L)]
                flat = (r << shift) | c
                m = s == 0.0
                pos = plsc.cumsum(m.astype(jnp.int32))
                plsc.store_scatter(zidx_v, [cv + pos - 1], flat, mask=m)
                return cv + plsc.all_reduce_population_count(m)

            cnt = cnt_vec[0]

            @pl.when(cnt > 0)
            def _():
                pad = jnp.full((_L,), zidx_v[pl.ds(0, _L)][0], jnp.int32)
                for k in range(_SBW // _L):
                    zidx_v[pl.ds(cnt + k * _L, _L)] = pad
                nb = (cnt + _SBW - 1) // _SBW
                nb2 = nb // 2

                def blk2(j, c2):
                    fire(2 * j, ssem0)
                    fire(2 * j + 1, ssem1)
                    return c2

                lax.fori_loop(0, nb2, blk2, 0)

                @pl.when(nb % 2 == 1)
                def _():
                    fire(nb - 1, ssem0)

                def drn2(j, c2):
                    drain1(ssem0)
                    drain1(ssem1)
                    return c2

                lax.fori_loop(0, nb2, drn2, 0)

                @pl.when(nb % 2 == 1)
                def _():
                    drain1(ssem0)

    return scatter_zeros


def kernel(input, new_sample, weight_rows, weight_cols, sample):
    n = input.shape[0]
    nnz = weight_rows.shape[0]
    out_ref = jax.new_ref(jnp.reshape(input, (-1,)))
    _build(nnz, n)(out_ref, weight_rows, weight_cols, sample)
    return jnp.reshape(out_ref[...], (n, n))
